# baseline (speedup 1.0000x reference)
import jax
import jax.numpy as jnp
from jax.experimental import pallas as pl
from jax.experimental.pallas import tpu as pltpu

B, N, D = 4, 2048, 256
TOPK = 16
R = 256
NBLK = N // R
NLANE = 128
NCHUNK = N // NLANE


def _fused_kernel(x_ref, o_ref, n_scr, t_scr, s_scr):
    p = pl.program_id(1)
    i = pl.program_id(2)

    @pl.when((p == 0) & (i == 0))
    def _normalize():
        v = x_ref[0]
        nrm = jnp.sqrt(jnp.sum(v * v, axis=-1, keepdims=True))
        n_scr[...] = v / jnp.maximum(nrm, 1e-12)

    @pl.when(p == 0)
    def _thresh():
        nb = n_scr[pl.ds(i * R, R), :]
        s = jax.lax.dot_general(nb, n_scr[...], (((1,), (1,)), ((), ())),
                                preferred_element_type=jnp.float32)
        s_scr[pl.ds(i * R, R), :] = s
        neg = jnp.full((R, NLANE), -jnp.inf, jnp.float32)
        h1 = h2 = h3 = h4 = neg
        for ci in range(NCHUNK):
            v = s[:, ci * NLANE:(ci + 1) * NLANE]
            m1 = jnp.maximum(h1, v); v = jnp.minimum(h1, v)
            m2 = jnp.maximum(h2, v); v = jnp.minimum(h2, v)
            m3 = jnp.maximum(h3, v); v = jnp.minimum(h3, v)
            m4 = jnp.maximum(h4, v)
            h1, h2, h3, h4 = m1, m2, m3, m4
        c = jnp.concatenate([h1, h2, h3, h4], axis=1)
        m = jnp.max(h1, axis=1, keepdims=True)
        for _ in range(TOPK - 1):
            m = jnp.max(jnp.where(c < m, c, -jnp.inf), axis=1, keepdims=True)
        cnt = jnp.sum(jnp.where(s >= m, 1.0, 0.0), axis=1, keepdims=True)
        t_scr[0, pl.ds(i * R, R)] = m[:, 0]

        @pl.when(jnp.any(cnt > TOPK + 0.5))
        def _fallback():
            mf = jnp.max(s, axis=1, keepdims=True)
            for _ in range(TOPK - 1):
                mf = jnp.max(jnp.where(s < mf, s, -jnp.inf),
                             axis=1, keepdims=True)
            t_scr[0, pl.ds(i * R, R)] = mf[:, 0]

    @pl.when(p == 1)
    def _adj():
        s = s_scr[pl.ds(i * R, R), :]
        t_col = t_scr[0, :]
        t_row = t_scr[0, pl.ds(i * R, R)]
        tmin = jnp.minimum(t_row[:, None], t_col[None, :])
        adj = jnp.where(s >= tmin, 1.0, 0.0)
        rows = jax.lax.broadcasted_iota(jnp.int32, (R, N), 0) + i * R
        cols = jax.lax.broadcasted_iota(jnp.int32, (R, N), 1)
        o_ref[0] = jnp.where(rows == cols, 1.0, adj)


@jax.jit
def kernel(x):
    return pl.pallas_call(
        _fused_kernel,
        grid=(B, 2, NBLK),
        in_specs=[pl.BlockSpec((1, N, D), lambda b, p, i: (b, 0, 0))],
        out_specs=pl.BlockSpec((1, R, N), lambda b, p, i: (b, i * p, 0)),
        out_shape=jax.ShapeDtypeStruct((B, N, N), jnp.float32),
        scratch_shapes=[
            pltpu.VMEM((N, D), jnp.float32),
            pltpu.VMEM((1, N), jnp.float32),
            pltpu.VMEM((N, N), jnp.float32),
        ],
    )(x)


# speedup vs baseline: 1.3786x; 1.3786x over previous
import jax
import jax.numpy as jnp
from jax.experimental import pallas as pl
from jax.experimental.pallas import tpu as pltpu

B, N, D = 4, 2048, 256
TOPK = 16
R = 1024
NBLK = N // R
NLANE = 128
NCHUNK = N // NLANE


def _fused_kernel(x_ref, o_ref, n_scr, t_scr, s_scr):
    p = pl.program_id(1)
    i = pl.program_id(2)

    @pl.when((p == 0) & (i == 0))
    def _normalize():
        v = x_ref[0]
        nrm = jnp.sqrt(jnp.sum(v * v, axis=-1, keepdims=True))
        n_scr[...] = v / jnp.maximum(nrm, 1e-12)

    @pl.when(p == 0)
    def _thresh():
        nb = n_scr[pl.ds(i * R, R), :]
        s = jax.lax.dot_general(nb, n_scr[...], (((1,), (1,)), ((), ())),
                                preferred_element_type=jnp.float32)
        s_scr[pl.ds(i * R, R), :] = s
        neg = jnp.full((R, NLANE), -jnp.inf, jnp.float32)
        h1 = h2 = h3 = h4 = neg
        for ci in range(NCHUNK):
            v = s[:, ci * NLANE:(ci + 1) * NLANE]
            m1 = jnp.maximum(h1, v); v = jnp.minimum(h1, v)
            m2 = jnp.maximum(h2, v); v = jnp.minimum(h2, v)
            m3 = jnp.maximum(h3, v); v = jnp.minimum(h3, v)
            m4 = jnp.maximum(h4, v)
            h1, h2, h3, h4 = m1, m2, m3, m4
        c = jnp.concatenate([h1, h2, h3, h4], axis=1)
        m = jnp.max(h1, axis=1, keepdims=True)
        for _ in range(TOPK - 1):
            m = jnp.max(jnp.where(c < m, c, -jnp.inf), axis=1, keepdims=True)
        cnt = jnp.sum(jnp.where(s >= m, 1.0, 0.0), axis=1, keepdims=True)
        t_scr[0, pl.ds(i * R, R)] = m[:, 0]

        @pl.when(jnp.any(cnt > TOPK + 0.5))
        def _fallback():
            mf = jnp.max(s, axis=1, keepdims=True)
            for _ in range(TOPK - 1):
                mf = jnp.max(jnp.where(s < mf, s, -jnp.inf),
                             axis=1, keepdims=True)
            t_scr[0, pl.ds(i * R, R)] = mf[:, 0]

    @pl.when(p == 1)
    def _adj():
        s = s_scr[pl.ds(i * R, R), :]
        t_col = t_scr[0, :]
        t_row = t_scr[0, pl.ds(i * R, R)]
        tmin = jnp.minimum(t_row[:, None], t_col[None, :])
        adj = jnp.where(s >= tmin, 1.0, 0.0)
        rows = jax.lax.broadcasted_iota(jnp.int32, (R, N), 0) + i * R
        cols = jax.lax.broadcasted_iota(jnp.int32, (R, N), 1)
        o_ref[0] = jnp.where(rows == cols, 1.0, adj)


@jax.jit
def kernel(x):
    return pl.pallas_call(
        _fused_kernel,
        grid=(B, 2, NBLK),
        in_specs=[pl.BlockSpec((1, N, D), lambda b, p, i: (b, 0, 0))],
        out_specs=pl.BlockSpec((1, R, N), lambda b, p, i: (b, i * p, 0)),
        out_shape=jax.ShapeDtypeStruct((B, N, N), jnp.float32),
        scratch_shapes=[
            pltpu.VMEM((N, D), jnp.float32),
            pltpu.VMEM((1, N), jnp.float32),
            pltpu.VMEM((N, N), jnp.float32),
        ],
    )(x)


# speedup vs baseline: 1.4430x; 1.0467x over previous
import jax
import jax.numpy as jnp
from jax.experimental import pallas as pl
from jax.experimental.pallas import tpu as pltpu

B, N, D = 4, 2048, 256
TOPK = 16
R = 1024
NBLK = N // R
NLANE = 128
NCHUNK = N // NLANE


def _fused_kernel(x_ref, o_ref, n_scr, t_scr, s_scr):
    p = pl.program_id(1)
    i = pl.program_id(2)

    @pl.when((p == 0) & (i == 0))
    def _normalize():
        v = x_ref[0]
        nrm = jnp.sqrt(jnp.sum(v * v, axis=-1, keepdims=True))
        n_scr[...] = v / jnp.maximum(nrm, 1e-12)

    @pl.when(p == 0)
    def _thresh():
        nb = n_scr[pl.ds(i * R, R), :]
        s = jax.lax.dot_general(nb, n_scr[...], (((1,), (1,)), ((), ())),
                                preferred_element_type=jnp.float32)
        s_scr[pl.ds(i * R, R), :] = s
        sl = [s[:, ci * NLANE:(ci + 1) * NLANE] for ci in range(NCHUNK)]

        def ce(a, b):
            return jnp.maximum(a, b), jnp.minimum(a, b)

        groups = []
        for g in range(4):
            a, b2, c2, d2 = sl[4 * g:4 * g + 4]
            a, b2 = ce(a, b2)
            c2, d2 = ce(c2, d2)
            a, c2 = ce(a, c2)
            b2, d2 = ce(b2, d2)
            b2, c2 = ce(b2, c2)
            groups.append((a, b2, c2, d2))

        def merge(A, Bl):
            tops = tuple(jnp.maximum(A[j], Bl[3 - j]) for j in range(4))
            disc = tuple(jnp.minimum(A[j], Bl[3 - j]) for j in range(4))
            return tops, disc

        def clean(M):
            a, b2, c2, d2 = M
            a, c2 = ce(a, c2)
            b2, d2 = ce(b2, d2)
            a, b2 = ce(a, b2)
            c2, d2 = ce(c2, d2)
            return (a, b2, c2, d2)

        m1t, d1 = merge(groups[0], groups[1])
        m2t, d2_ = merge(groups[2], groups[3])
        ftop, d3 = merge(clean(m1t), clean(m2t))
        h5 = d1[0]
        for dd in d1[1:] + d2_ + d3:
            h5 = jnp.maximum(h5, dd)
        c = jnp.concatenate(ftop, axis=1)
        m = jnp.max(jnp.maximum(jnp.maximum(ftop[0], ftop[1]),
                                jnp.maximum(ftop[2], ftop[3])),
                    axis=1, keepdims=True)
        for _ in range(TOPK - 1):
            m = jnp.max(jnp.where(c < m, c, -jnp.inf), axis=1, keepdims=True)
        t_scr[0, pl.ds(i * R, R)] = m[:, 0]

        @pl.when(jnp.any(h5 >= m))
        def _fallback():
            mf = jnp.max(s, axis=1, keepdims=True)
            for _ in range(TOPK - 1):
                mf = jnp.max(jnp.where(s < mf, s, -jnp.inf),
                             axis=1, keepdims=True)
            t_scr[0, pl.ds(i * R, R)] = mf[:, 0]

    @pl.when(p == 1)
    def _adj():
        s = s_scr[pl.ds(i * R, R), :]
        t_col = t_scr[0, :]
        t_row = t_scr[0, pl.ds(i * R, R)]
        tmin = jnp.minimum(t_row[:, None], t_col[None, :])
        adj = jnp.where(s >= tmin, 1.0, 0.0)
        rows = jax.lax.broadcasted_iota(jnp.int32, (R, N), 0) + i * R
        cols = jax.lax.broadcasted_iota(jnp.int32, (R, N), 1)
        o_ref[0] = jnp.where(rows == cols, 1.0, adj)


@jax.jit
def kernel(x):
    return pl.pallas_call(
        _fused_kernel,
        grid=(B, 2, NBLK),
        in_specs=[pl.BlockSpec((1, N, D), lambda b, p, i: (b, 0, 0))],
        out_specs=pl.BlockSpec((1, R, N), lambda b, p, i: (b, i * p, 0)),
        out_shape=jax.ShapeDtypeStruct((B, N, N), jnp.float32),
        scratch_shapes=[
            pltpu.VMEM((N, D), jnp.float32),
            pltpu.VMEM((1, N), jnp.float32),
            pltpu.VMEM((N, N), jnp.float32),
        ],
    )(x)
